# baseline (speedup 1.0000x reference)
import functools
import jax
import jax.numpy as jnp
from jax import lax
from jax.experimental import pallas as pl
from jax.experimental.pallas import tpu as pltpu
from jax.experimental.pallas import tpu_sc as plsc

_NC = 2
_NS = 16
_NW = _NC * _NS
_PG = 4


def _gather(weight, idx3, b, f):
    _, n_chunks, chunk_idx = idx3.shape
    D = weight.shape[1]
    planes_per_w = b // _NW
    NBUF = 4
    n_main = n_chunks // NBUF

    mesh = plsc.VectorSubcoreMesh(core_axis_name="c", subcore_axis_name="s")

    @functools.partial(
        pl.kernel,
        mesh=mesh,
        out_type=jax.ShapeDtypeStruct((b, f, D), jnp.float32),
        scratch_types=[
            pltpu.VMEM((n_chunks, chunk_idx), jnp.int32),
        ]
        + [pltpu.VMEM((_PG * f, D), jnp.float32)] * NBUF
        + [pltpu.SemaphoreType.DMA] * (2 * NBUF),
    )
    def k(idx_hbm, table_hbm, out_hbm, idx_v, *bufs_sems):
        bufs = bufs_sems[:NBUF]
        gsem = bufs_sems[NBUF : 2 * NBUF]
        wsem = bufs_sems[2 * NBUF :]
        wid = lax.axis_index("s") * _NC + lax.axis_index("c")
        plane0 = wid * planes_per_w
        pltpu.sync_copy(idx_hbm.at[wid], idx_v)

        def g_start(c, bf):
            pltpu.async_copy(table_hbm.at[idx_v.at[c]], bufs[bf], gsem[bf])

        def g_wait(bf):
            pltpu.make_async_copy(table_hbm.at[idx_v.at[0]], bufs[bf], gsem[bf]).wait()

        def w_start(c, bf):
            for p in range(_PG):
                pltpu.async_copy(
                    bufs[bf].at[pl.ds(p * f, f)],
                    out_hbm.at[plane0 + c * _PG + p],
                    wsem[bf],
                )

        def w_wait(bf):
            for p in range(_PG):
                pltpu.make_async_copy(
                    bufs[bf].at[pl.ds(p * f, f)], out_hbm.at[plane0], wsem[bf]
                ).wait()

        for bf in range(NBUF):
            g_start(bf, bf)

        def body(g, carry):
            c0 = g * NBUF
            for bf in range(NBUF):
                g_wait(bf)
                w_start(c0 + bf, bf)
            for bf in range(NBUF):
                nc = c0 + bf + NBUF

                @pl.when(nc < n_chunks)
                def _(nc=nc, bf=bf):
                    w_wait(bf)
                    g_start(nc, bf)

            return carry

        lax.fori_loop(0, n_main, body, 0)
        for bf in range(n_chunks - n_main * NBUF):
            g_wait(bf)
            w_start(n_main * NBUF + bf, bf)
        for bf in range(NBUF):
            w_wait(bf)

    return k(idx3, weight)


_KSPLIT = 4


def kernel(weight, indices):
    b, f = indices.shape
    bk = b // _KSPLIT
    parts = []
    for k in range(_KSPLIT):
        idx = indices[k * bk : (k + 1) * bk].reshape(-1).astype(jnp.int32)
        idx3 = idx.reshape(_NW, -1, _PG * f)
        parts.append(_gather(weight, idx3, bk, f))
    return jnp.concatenate(parts, axis=0)


# speedup vs baseline: 1.1602x; 1.1602x over previous
import functools
import jax
import jax.numpy as jnp
from jax import lax
from jax.experimental import pallas as pl
from jax.experimental.pallas import tpu as pltpu
from jax.experimental.pallas import tpu_sc as plsc

_NC = 2
_NS = 16
_NW = _NC * _NS
_PG = 4


def _gather(weight, idx3, b, f):
    _, n_chunks, chunk_idx = idx3.shape
    D = weight.shape[1]
    planes_per_w = b // _NW
    NBUF = 4
    n_main = n_chunks // NBUF

    mesh = plsc.VectorSubcoreMesh(core_axis_name="c", subcore_axis_name="s")

    @functools.partial(
        pl.kernel,
        mesh=mesh,
        out_type=jax.ShapeDtypeStruct((b, f, D), jnp.float32),
        scratch_types=[
            pltpu.VMEM((n_chunks, chunk_idx), jnp.int32),
        ]
        + [pltpu.VMEM((_PG * f, D), jnp.float32)] * NBUF
        + [pltpu.SemaphoreType.DMA] * (2 * NBUF),
    )
    def k(idx_hbm, table_hbm, out_hbm, idx_v, *bufs_sems):
        bufs = bufs_sems[:NBUF]
        gsem = bufs_sems[NBUF : 2 * NBUF]
        wsem = bufs_sems[2 * NBUF :]
        wid = lax.axis_index("s") * _NC + lax.axis_index("c")
        plane0 = wid * planes_per_w
        pltpu.sync_copy(idx_hbm.at[wid], idx_v)

        def g_start(c, bf):
            pltpu.async_copy(table_hbm.at[idx_v.at[c]], bufs[bf], gsem[bf])

        def g_wait(bf):
            pltpu.make_async_copy(table_hbm.at[idx_v.at[0]], bufs[bf], gsem[bf]).wait()

        def w_start(c, bf):
            for p in range(_PG):
                pltpu.async_copy(
                    bufs[bf].at[pl.ds(p * f, f)],
                    out_hbm.at[plane0 + c * _PG + p],
                    wsem[bf],
                )

        def w_wait(bf):
            for p in range(_PG):
                pltpu.make_async_copy(
                    bufs[bf].at[pl.ds(p * f, f)], out_hbm.at[plane0], wsem[bf]
                ).wait()

        for bf in range(NBUF):
            g_start(bf, bf)

        def body(g, carry):
            c0 = g * NBUF
            for bf in range(NBUF):
                g_wait(bf)
                w_start(c0 + bf, bf)
            for bf in range(NBUF):
                nc = c0 + bf + NBUF

                @pl.when(nc < n_chunks)
                def _(nc=nc, bf=bf):
                    w_wait(bf)
                    g_start(nc, bf)

            return carry

        lax.fori_loop(0, n_main, body, 0)
        for bf in range(n_chunks - n_main * NBUF):
            g_wait(bf)
            w_start(n_main * NBUF + bf, bf)
        for bf in range(NBUF):
            w_wait(bf)

    return k(idx3, weight)


_KSPLIT = 2


def kernel(weight, indices):
    b, f = indices.shape
    bk = b // _KSPLIT
    parts = []
    for k in range(_KSPLIT):
        idx = indices[k * bk : (k + 1) * bk].reshape(-1).astype(jnp.int32)
        idx3 = idx.reshape(_NW, -1, _PG * f)
        parts.append(_gather(weight, idx3, bk, f))
    return jnp.concatenate(parts, axis=0)


# speedup vs baseline: 3.0326x; 2.6139x over previous
import functools
import jax
import jax.numpy as jnp
from jax import lax
from jax.experimental import pallas as pl
from jax.experimental.pallas import tpu as pltpu
from jax.experimental.pallas import tpu_sc as plsc

_NC = 2
_NS = 16
_NW = _NC * _NS
_CHUNK = 128


def _gather(weight, idx):
    B = idx.shape[0]
    D = weight.shape[1]
    per_w = B // _NW
    n_chunks = per_w // _CHUNK
    NBUF = 4
    n_main = n_chunks // NBUF

    mesh = plsc.VectorSubcoreMesh(core_axis_name="c", subcore_axis_name="s")

    @functools.partial(
        pl.kernel,
        mesh=mesh,
        out_type=jax.ShapeDtypeStruct((B, D), jnp.float32),
        scratch_types=[
            pltpu.VMEM((per_w,), jnp.int32),
        ]
        + [pltpu.VMEM((_CHUNK, D), jnp.float32)] * NBUF
        + [pltpu.SemaphoreType.DMA] * (2 * NBUF),
    )
    def k(idx_hbm, table_hbm, out_hbm, idx_v, *bufs_sems):
        bufs = bufs_sems[:NBUF]
        gsem = bufs_sems[NBUF : 2 * NBUF]
        wsem = bufs_sems[2 * NBUF :]
        wid = lax.axis_index("s") * _NC + lax.axis_index("c")
        base = wid * per_w
        pltpu.sync_copy(idx_hbm.at[pl.ds(base, per_w)], idx_v)

        def g_start(c, bf):
            pltpu.async_copy(
                table_hbm.at[idx_v.at[pl.ds(c * _CHUNK, _CHUNK)]], bufs[bf], gsem[bf]
            )

        def g_wait(bf):
            pltpu.make_async_copy(
                table_hbm.at[idx_v.at[pl.ds(0, _CHUNK)]], bufs[bf], gsem[bf]
            ).wait()

        def w_start(c, bf):
            pltpu.async_copy(
                bufs[bf], out_hbm.at[pl.ds(base + c * _CHUNK, _CHUNK)], wsem[bf]
            )

        def w_wait(bf):
            pltpu.make_async_copy(
                bufs[bf], out_hbm.at[pl.ds(base, _CHUNK)], wsem[bf]
            ).wait()

        for bf in range(NBUF):
            g_start(bf, bf)

        def body(g, carry):
            c0 = g * NBUF
            for bf in range(NBUF):
                g_wait(bf)
                w_start(c0 + bf, bf)
            for bf in range(NBUF):
                nc = c0 + bf + NBUF

                @pl.when(nc < n_chunks)
                def _(nc=nc, bf=bf):
                    w_wait(bf)
                    g_start(nc, bf)

            return carry

        lax.fori_loop(0, n_main, body, 0)
        for bf in range(n_chunks - n_main * NBUF):
            g_wait(bf)
            w_start(n_main * NBUF + bf, bf)
        for bf in range(NBUF):
            w_wait(bf)

    return k(idx, weight)


def kernel(weight, indices):
    b, f = indices.shape
    d = weight.shape[1]
    idx = indices.T.reshape(-1).astype(jnp.int32)
    out_flat = _gather(weight, idx)
    return out_flat.reshape(f, b, d).transpose(1, 0, 2)


# speedup vs baseline: 3.1384x; 1.0349x over previous
import functools
import jax
import jax.numpy as jnp
from jax import lax
from jax.experimental import pallas as pl
from jax.experimental.pallas import tpu as pltpu
from jax.experimental.pallas import tpu_sc as plsc

_NC = 2
_NS = 16
_NW = _NC * _NS
_CHUNK = 128


def _gather(weight, idx):
    B = idx.shape[0]
    D = weight.shape[1]
    per_w = B // _NW
    n_chunks = per_w // _CHUNK
    NBUF = 6
    n_main = n_chunks // NBUF

    mesh = plsc.VectorSubcoreMesh(core_axis_name="c", subcore_axis_name="s")

    @functools.partial(
        pl.kernel,
        mesh=mesh,
        out_type=jax.ShapeDtypeStruct((B, D), jnp.float32),
        scratch_types=[
            pltpu.VMEM((per_w,), jnp.int32),
        ]
        + [pltpu.VMEM((_CHUNK, D), jnp.float32)] * NBUF
        + [pltpu.SemaphoreType.DMA] * (2 * NBUF),
    )
    def k(idx_hbm, table_hbm, out_hbm, idx_v, *bufs_sems):
        bufs = bufs_sems[:NBUF]
        gsem = bufs_sems[NBUF : 2 * NBUF]
        wsem = bufs_sems[2 * NBUF :]
        wid = lax.axis_index("s") * _NC + lax.axis_index("c")
        base = wid * per_w
        pltpu.sync_copy(idx_hbm.at[pl.ds(base, per_w)], idx_v)

        def g_start(c, bf):
            pltpu.async_copy(
                table_hbm.at[idx_v.at[pl.ds(c * _CHUNK, _CHUNK)]], bufs[bf], gsem[bf]
            )

        def g_wait(bf):
            pltpu.make_async_copy(
                table_hbm.at[idx_v.at[pl.ds(0, _CHUNK)]], bufs[bf], gsem[bf]
            ).wait()

        def w_start(c, bf):
            pltpu.async_copy(
                bufs[bf], out_hbm.at[pl.ds(base + c * _CHUNK, _CHUNK)], wsem[bf]
            )

        def w_wait(bf):
            pltpu.make_async_copy(
                bufs[bf], out_hbm.at[pl.ds(base, _CHUNK)], wsem[bf]
            ).wait()

        for bf in range(NBUF):
            g_start(bf, bf)

        def body(g, carry):
            c0 = g * NBUF
            for bf in range(NBUF):
                g_wait(bf)
                w_start(c0 + bf, bf)
            for bf in range(NBUF):
                nc = c0 + bf + NBUF

                @pl.when(nc < n_chunks)
                def _(nc=nc, bf=bf):
                    w_wait(bf)
                    g_start(nc, bf)

            return carry

        lax.fori_loop(0, n_main, body, 0)
        for bf in range(n_chunks - n_main * NBUF):
            g_wait(bf)
            w_start(n_main * NBUF + bf, bf)
        for bf in range(NBUF):
            w_wait(bf)

    return k(idx, weight)


def kernel(weight, indices):
    b, f = indices.shape
    d = weight.shape[1]
    idx = indices.T.reshape(-1).astype(jnp.int32)
    out_flat = _gather(weight, idx)
    return out_flat.reshape(f, b, d).transpose(1, 0, 2)
